# baseline (speedup 1.0000x reference)
import functools

import jax
import jax.numpy as jnp
from jax import lax
from jax.experimental import pallas as pl
from jax.experimental.pallas import tpu as pltpu
from jax.experimental.pallas import tpu_sc as plsc

N = 10000
D = 128
H = 128
T = 10
G = 64
L = 3
BN_EPS = 1e-5

NC = 1
NS = 16
NW = NC * NS
EDGE_CHUNK = 96
NB = 3
N_ACC = 10112
ZROWS = N_ACC // NS


def _edge_agg(h, src_pad, dst_pad, zeros_blk):
    e_pad = src_pad.shape[0]
    e_per_w = e_pad // NW
    n_super = e_per_w // (EDGE_CHUNK * NB)
    out_rows = N_ACC // NS
    mesh = plsc.VectorSubcoreMesh(core_axis_name="c", subcore_axis_name="s", num_cores=NC)

    @functools.partial(
        pl.kernel,
        out_type=jax.ShapeDtypeStruct((NC, N_ACC, H), jnp.float32),
        mesh=mesh,
        scratch_types=(
            [pltpu.VMEM((EDGE_CHUNK,), jnp.int32)] * NB
            + [pltpu.VMEM((EDGE_CHUNK,), jnp.int32)] * NB
            + [pltpu.VMEM((EDGE_CHUNK, H), jnp.float32)] * NB
            + [
                pltpu.VMEM_SHARED((N_ACC, H), jnp.float32),
                pltpu.SemaphoreType.DMA((NB,)),
                pltpu.SemaphoreType.DMA((NB,)),
                pltpu.SemaphoreType.DMA((NB,)),
            ]
        ),
    )
    def k(h_hbm, src_hbm, dst_hbm, z_hbm, out_hbm,
          s0, s1, s2, d0, d1, d2, r0, r1, r2, acc,
          isem, gsem, ssem):
        src_v = [s0, s1, s2]
        dst_v = [d0, d1, d2]
        rows_v = [r0, r1, r2]
        cid = lax.axis_index("c")
        sid = lax.axis_index("s")
        wid = sid * NC + cid
        pltpu.sync_copy(z_hbm, acc.at[pl.ds(sid * ZROWS, ZROWS)])
        plsc.subcore_barrier()
        base = wid * e_per_w

        def body(j0, carry):
            off0 = base + j0 * (EDGE_CHUNK * NB)
            idx_cp = []
            for b in range(NB):
                off = off0 + b * EDGE_CHUNK
                c1 = pltpu.async_copy(
                    src_hbm.at[pl.ds(off, EDGE_CHUNK)], src_v[b], isem.at[b])
                c2 = pltpu.async_copy(
                    dst_hbm.at[pl.ds(off, EDGE_CHUNK)], dst_v[b], isem.at[b])
                idx_cp.append((c1, c2))
            g = [None] * NB
            s = [None] * NB
            for b in range(NB):
                idx_cp[b][0].wait()
                idx_cp[b][1].wait()
                g[b] = pltpu.async_copy(
                    h_hbm.at[src_v[b]], rows_v[b], gsem.at[b])
                if b > 0:
                    g[b - 1].wait()
                    s[b - 1] = pltpu.async_copy(
                        rows_v[b - 1], acc.at[dst_v[b - 1]],
                        ssem.at[b - 1], add=True)
            g[NB - 1].wait()
            s[NB - 1] = pltpu.async_copy(
                rows_v[NB - 1], acc.at[dst_v[NB - 1]],
                ssem.at[NB - 1], add=True)
            for b in range(NB):
                s[b].wait()
            return carry

        lax.fori_loop(0, n_super, body, 0)
        plsc.subcore_barrier()
        pltpu.sync_copy(
            acc.at[pl.ds(sid * out_rows, out_rows)],
            out_hbm.at[cid, pl.ds(sid * out_rows, out_rows)],
        )

    return k(h, src_pad, dst_pad, zeros_blk)


def _vspec():
    return pl.BlockSpec(memory_space=pltpu.VMEM)


def _layer0_body(x_ref, w1_ref, a1_ref, c1_ref, w2_ref, a2_ref, c2_ref,
                 wl_ref, bl_ref, batch_ref, h_ref, out_ref):
    x = x_ref[...]
    h = jnp.maximum(
        jnp.dot(x, w1_ref[...], precision="highest") * a1_ref[...] + c1_ref[...], 0.0)
    h = jnp.maximum(
        jnp.dot(h, w2_ref[...], precision="highest") * a2_ref[...] + c2_ref[...], 0.0)
    h_ref[...] = h
    iota = lax.broadcasted_iota(jnp.int32, (G, N), 0)
    onehot = (batch_ref[...] == iota).astype(jnp.float32)
    pooled = jnp.dot(onehot, h, precision="highest")
    counts = jnp.sum(onehot, axis=1, keepdims=True)
    out_ref[...] = (jnp.dot(pooled, wl_ref[...], precision="highest")
                    + counts * bl_ref[...])


def _layerl_body(h_in_ref, agg_ref, epsp_ref, w1_ref, a1_ref, c1_ref,
                 w2_ref, a2_ref, c2_ref, wl_ref, bl_ref, batch_ref, acc_ref,
                 h_ref, out_ref):
    hin = h_in_ref[...] * epsp_ref[0, 0] + agg_ref[0, :N, :]
    if agg_ref.shape[0] > 1:
        hin = hin + agg_ref[1, :N, :]
    h = jnp.maximum(
        jnp.dot(hin, w1_ref[...], precision="highest") * a1_ref[...] + c1_ref[...], 0.0)
    h = jnp.maximum(
        jnp.dot(h, w2_ref[...], precision="highest") * a2_ref[...] + c2_ref[...], 0.0)
    h_ref[...] = h
    iota = lax.broadcasted_iota(jnp.int32, (G, N), 0)
    onehot = (batch_ref[...] == iota).astype(jnp.float32)
    pooled = jnp.dot(onehot, h, precision="highest")
    out_ref[...] = (acc_ref[...]
                    + jnp.dot(pooled, wl_ref[...], precision="highest")
                    + bl_ref[...])


def _fold_bn(p):
    inv = 1.0 / jnp.sqrt(1.0 + BN_EPS)
    a1 = (p["g1"] * inv)[None, :]
    c1 = (p["b1"] * p["g1"] * inv + p["be1"])[None, :]
    a2 = (p["g2"] * inv)[None, :]
    c2 = (p["b2"] * p["g2"] * inv + p["be2"])[None, :]
    return p["W1"], a1, c1, p["W2"], a2, c2


def _layer0(x, params, batch2):
    w1, a1, c1, w2, a2, c2 = _fold_bn(params["first_h"])
    lin = params["linears"][0]
    return pl.pallas_call(
        _layer0_body,
        out_shape=[
            jax.ShapeDtypeStruct((N, H), jnp.float32),
            jax.ShapeDtypeStruct((G, T), jnp.float32),
        ],
        in_specs=[_vspec()] * 10,
        out_specs=[_vspec(), _vspec()],
    )(x, w1, a1, c1, w2, a2, c2, lin["W"], lin["b"][None, :], batch2)


def _layerl(h, agg, out_acc, conv, lin, batch2):
    w1, a1, c1, w2, a2, c2 = _fold_bn(conv["nn"])
    epsp = (1.0 + conv["eps"]).reshape(1, 1).astype(jnp.float32)
    return pl.pallas_call(
        _layerl_body,
        out_shape=[
            jax.ShapeDtypeStruct((N, H), jnp.float32),
            jax.ShapeDtypeStruct((G, T), jnp.float32),
        ],
        in_specs=([_vspec(), _vspec(), pl.BlockSpec(memory_space=pltpu.SMEM)]
                  + [_vspec()] * 10),
        out_specs=[_vspec(), _vspec()],
    )(h, agg, epsp, w1, a1, c1, w2, a2, c2, lin["W"], lin["b"][None, :],
      batch2, out_acc)


def kernel(x, edge_index, batch, params):
    e = edge_index.shape[1]
    quantum = NW * EDGE_CHUNK * NB
    e_pad = ((e + quantum - 1) // quantum) * quantum
    pad = e_pad - e
    src_pad = jnp.concatenate(
        [edge_index[0], jnp.zeros((pad,), jnp.int32)])
    dst_pad = jnp.concatenate(
        [edge_index[1], jnp.full((pad,), N_ACC - 1, jnp.int32)])
    zeros_blk = jnp.zeros((ZROWS, H), jnp.float32)
    batch2 = batch[None, :]

    h, out = _layer0(x, params, batch2)
    for l in range(L):
        agg = _edge_agg(h, src_pad, dst_pad, zeros_blk)
        h, out = _layerl(h, agg, out, params["convs"][l],
                         params["linears"][l + 1], batch2)
    return jax.nn.sigmoid(out)


# speedup vs baseline: 1.4403x; 1.4403x over previous
import functools

import jax
import jax.numpy as jnp
from jax import lax
from jax.experimental import pallas as pl
from jax.experimental.pallas import tpu as pltpu
from jax.experimental.pallas import tpu_sc as plsc

N = 10000
D = 128
H = 128
T = 10
G = 64
L = 3
BN_EPS = 1e-5

NC = 2
NS = 16
NW = NC * NS
EDGE_CHUNK = 96
NB = 3
N_ACC = 10112
ZROWS = N_ACC // NS


def _edge_agg(h, src_pad, dst_pad, zeros_blk):
    e_pad = src_pad.shape[0]
    e_per_w = e_pad // NW
    n_super = e_per_w // (EDGE_CHUNK * NB)
    out_rows = N_ACC // NS
    mesh = plsc.VectorSubcoreMesh(core_axis_name="c", subcore_axis_name="s", num_cores=NC)

    @functools.partial(
        pl.kernel,
        out_type=jax.ShapeDtypeStruct((NC, N_ACC, H), jnp.float32),
        mesh=mesh,
        scratch_types=(
            [pltpu.VMEM((EDGE_CHUNK,), jnp.int32)] * NB
            + [pltpu.VMEM((EDGE_CHUNK,), jnp.int32)] * NB
            + [pltpu.VMEM((EDGE_CHUNK, H), jnp.float32)] * NB
            + [
                pltpu.VMEM_SHARED((N_ACC, H), jnp.float32),
                pltpu.SemaphoreType.DMA((NB,)),
                pltpu.SemaphoreType.DMA((NB,)),
                pltpu.SemaphoreType.DMA((NB,)),
            ]
        ),
    )
    def k(h_hbm, src_hbm, dst_hbm, z_hbm, out_hbm,
          s0, s1, s2, d0, d1, d2, r0, r1, r2, acc,
          isem, gsem, ssem):
        src_v = [s0, s1, s2]
        dst_v = [d0, d1, d2]
        rows_v = [r0, r1, r2]
        cid = lax.axis_index("c")
        sid = lax.axis_index("s")
        wid = sid * NC + cid
        pltpu.sync_copy(z_hbm, acc.at[pl.ds(sid * ZROWS, ZROWS)])
        plsc.subcore_barrier()
        base = wid * e_per_w

        def fire_idx(j0, b):
            off = base + j0 * (EDGE_CHUNK * NB) + b * EDGE_CHUNK
            pltpu.async_copy(
                src_hbm.at[pl.ds(off, EDGE_CHUNK)], src_v[b], isem.at[b])
            pltpu.async_copy(
                dst_hbm.at[pl.ds(off, EDGE_CHUNK)], dst_v[b], isem.at[b])

        def wait_idx(b):
            pltpu.make_async_copy(
                src_hbm.at[pl.ds(0, EDGE_CHUNK)], src_v[b], isem.at[b]).wait()
            pltpu.make_async_copy(
                dst_hbm.at[pl.ds(0, EDGE_CHUNK)], dst_v[b], isem.at[b]).wait()

        def wait_scat(b):
            pltpu.make_async_copy(
                rows_v[b], acc.at[dst_v[b]], ssem.at[b]).wait()

        for b in range(NB):
            fire_idx(0, b)

        def body(j0, carry):
            g = [None] * NB
            for b in range(NB):
                wait_idx(b)

                @pl.when(j0 > 0)
                def _drain():
                    wait_scat(b)

                g[b] = pltpu.async_copy(
                    h_hbm.at[src_v[b]], rows_v[b], gsem.at[b])
            for b in range(NB):
                g[b].wait()
                pltpu.async_copy(
                    rows_v[b], acc.at[dst_v[b]], ssem.at[b], add=True)

                @pl.when(j0 < n_super - 1)
                def _prefetch():
                    fire_idx(j0 + 1, b)

            return carry

        lax.fori_loop(0, n_super, body, 0)
        for b in range(NB):
            wait_scat(b)
        plsc.subcore_barrier()
        pltpu.sync_copy(
            acc.at[pl.ds(sid * out_rows, out_rows)],
            out_hbm.at[cid, pl.ds(sid * out_rows, out_rows)],
        )

    return k(h, src_pad, dst_pad, zeros_blk)


def _vspec():
    return pl.BlockSpec(memory_space=pltpu.VMEM)


def _layer0_body(x_ref, w1_ref, a1_ref, c1_ref, w2_ref, a2_ref, c2_ref,
                 wl_ref, bl_ref, batch_ref, h_ref, out_ref):
    x = x_ref[...]
    h = jnp.maximum(
        jnp.dot(x, w1_ref[...], precision="highest") * a1_ref[...] + c1_ref[...], 0.0)
    h = jnp.maximum(
        jnp.dot(h, w2_ref[...], precision="highest") * a2_ref[...] + c2_ref[...], 0.0)
    h_ref[...] = h
    iota = lax.broadcasted_iota(jnp.int32, (G, N), 0)
    onehot = (batch_ref[...] == iota).astype(jnp.float32)
    pooled = jnp.dot(onehot, h, precision="highest")
    counts = jnp.sum(onehot, axis=1, keepdims=True)
    out_ref[...] = (jnp.dot(pooled, wl_ref[...], precision="highest")
                    + counts * bl_ref[...])


def _layerl_body(h_in_ref, agg_ref, epsp_ref, w1_ref, a1_ref, c1_ref,
                 w2_ref, a2_ref, c2_ref, wl_ref, bl_ref, batch_ref, acc_ref,
                 h_ref, out_ref):
    hin = h_in_ref[...] * epsp_ref[0, 0] + agg_ref[0, :N, :]
    if agg_ref.shape[0] > 1:
        hin = hin + agg_ref[1, :N, :]
    h = jnp.maximum(
        jnp.dot(hin, w1_ref[...], precision="highest") * a1_ref[...] + c1_ref[...], 0.0)
    h = jnp.maximum(
        jnp.dot(h, w2_ref[...], precision="highest") * a2_ref[...] + c2_ref[...], 0.0)
    h_ref[...] = h
    iota = lax.broadcasted_iota(jnp.int32, (G, N), 0)
    onehot = (batch_ref[...] == iota).astype(jnp.float32)
    pooled = jnp.dot(onehot, h, precision="highest")
    out_ref[...] = (acc_ref[...]
                    + jnp.dot(pooled, wl_ref[...], precision="highest")
                    + bl_ref[...])


def _fold_bn(p):
    inv = 1.0 / jnp.sqrt(1.0 + BN_EPS)
    a1 = (p["g1"] * inv)[None, :]
    c1 = (p["b1"] * p["g1"] * inv + p["be1"])[None, :]
    a2 = (p["g2"] * inv)[None, :]
    c2 = (p["b2"] * p["g2"] * inv + p["be2"])[None, :]
    return p["W1"], a1, c1, p["W2"], a2, c2


def _layer0(x, params, batch2):
    w1, a1, c1, w2, a2, c2 = _fold_bn(params["first_h"])
    lin = params["linears"][0]
    return pl.pallas_call(
        _layer0_body,
        out_shape=[
            jax.ShapeDtypeStruct((N, H), jnp.float32),
            jax.ShapeDtypeStruct((G, T), jnp.float32),
        ],
        in_specs=[_vspec()] * 10,
        out_specs=[_vspec(), _vspec()],
    )(x, w1, a1, c1, w2, a2, c2, lin["W"], lin["b"][None, :], batch2)


def _layerl(h, agg, out_acc, conv, lin, batch2):
    w1, a1, c1, w2, a2, c2 = _fold_bn(conv["nn"])
    epsp = (1.0 + conv["eps"]).reshape(1, 1).astype(jnp.float32)
    return pl.pallas_call(
        _layerl_body,
        out_shape=[
            jax.ShapeDtypeStruct((N, H), jnp.float32),
            jax.ShapeDtypeStruct((G, T), jnp.float32),
        ],
        in_specs=([_vspec(), _vspec(), pl.BlockSpec(memory_space=pltpu.SMEM)]
                  + [_vspec()] * 10),
        out_specs=[_vspec(), _vspec()],
    )(h, agg, epsp, w1, a1, c1, w2, a2, c2, lin["W"], lin["b"][None, :],
      batch2, out_acc)


def kernel(x, edge_index, batch, params):
    e = edge_index.shape[1]
    quantum = NW * EDGE_CHUNK * NB
    e_pad = ((e + quantum - 1) // quantum) * quantum
    pad = e_pad - e
    src_pad = jnp.concatenate(
        [edge_index[0], jnp.zeros((pad,), jnp.int32)])
    dst_pad = jnp.concatenate(
        [edge_index[1], jnp.full((pad,), N_ACC - 1, jnp.int32)])
    zeros_blk = jnp.zeros((ZROWS, H), jnp.float32)
    batch2 = batch[None, :]

    h, out = _layer0(x, params, batch2)
    for l in range(L):
        agg = _edge_agg(h, src_pad, dst_pad, zeros_blk)
        h, out = _layerl(h, agg, out, params["convs"][l],
                         params["linears"][l + 1], batch2)
    return jax.nn.sigmoid(out)


# speedup vs baseline: 1.6716x; 1.1606x over previous
import functools

import jax
import jax.numpy as jnp
from jax import lax
from jax.experimental import pallas as pl
from jax.experimental.pallas import tpu as pltpu
from jax.experimental.pallas import tpu_sc as plsc

N = 10000
D = 128
H = 128
T = 10
G = 64
L = 3
BN_EPS = 1e-5

NC = 2
NS = 16
NW = NC * NS
EDGE_CHUNK = 96
NB = 3
S0 = 48
S1 = 22
N_ACC = 10112
ZROWS = N_ACC // NS


def _edge_agg(h, src_pad, dst_pad, zeros_blk):
    out_rows = N_ACC // NS
    per_tile0 = EDGE_CHUNK * NB * S0
    per_tile1 = EDGE_CHUNK * NB * S1
    mesh = plsc.VectorSubcoreMesh(core_axis_name="c", subcore_axis_name="s", num_cores=NC)

    @functools.partial(
        pl.kernel,
        out_type=jax.ShapeDtypeStruct((NC, N_ACC, H), jnp.float32),
        mesh=mesh,
        scratch_types=(
            [pltpu.VMEM((EDGE_CHUNK,), jnp.int32)] * NB
            + [pltpu.VMEM((EDGE_CHUNK,), jnp.int32)] * NB
            + [pltpu.VMEM((EDGE_CHUNK, H), jnp.float32)] * NB
            + [
                pltpu.VMEM_SHARED((N_ACC, H), jnp.float32),
                pltpu.SemaphoreType.DMA((NB,)),
                pltpu.SemaphoreType.DMA((NB,)),
                pltpu.SemaphoreType.DMA((NB,)),
            ]
        ),
    )
    def k(h_hbm, src_hbm, dst_hbm, z_hbm, out_hbm,
          s0, s1, s2, d0, d1, d2, r0, r1, r2, acc,
          isem, gsem, ssem):
        src_v = [s0, s1, s2]
        dst_v = [d0, d1, d2]
        rows_v = [r0, r1, r2]
        cid = lax.axis_index("c")
        sid = lax.axis_index("s")
        pltpu.sync_copy(z_hbm, acc.at[pl.ds(sid * ZROWS, ZROWS)])
        plsc.subcore_barrier()
        n_super = jnp.where(cid == 0, S0, S1)
        base = jnp.where(cid == 0, sid * per_tile0,
                         NS * per_tile0 + sid * per_tile1)

        def fire_idx(j0, b):
            off = base + j0 * (EDGE_CHUNK * NB) + b * EDGE_CHUNK
            pltpu.async_copy(
                src_hbm.at[pl.ds(off, EDGE_CHUNK)], src_v[b], isem.at[b])
            pltpu.async_copy(
                dst_hbm.at[pl.ds(off, EDGE_CHUNK)], dst_v[b], isem.at[b])

        def wait_idx(b):
            pltpu.make_async_copy(
                src_hbm.at[pl.ds(0, EDGE_CHUNK)], src_v[b], isem.at[b]).wait()
            pltpu.make_async_copy(
                dst_hbm.at[pl.ds(0, EDGE_CHUNK)], dst_v[b], isem.at[b]).wait()

        def wait_scat(b):
            pltpu.make_async_copy(
                rows_v[b], acc.at[dst_v[b]], ssem.at[b]).wait()

        for b in range(NB):
            fire_idx(0, b)

        def body(j0, carry):
            g = [None] * NB
            for b in range(NB):
                wait_idx(b)

                @pl.when(j0 > 0)
                def _drain():
                    wait_scat(b)

                g[b] = pltpu.async_copy(
                    h_hbm.at[src_v[b]], rows_v[b], gsem.at[b])
            for b in range(NB):
                g[b].wait()
                pltpu.async_copy(
                    rows_v[b], acc.at[dst_v[b]], ssem.at[b], add=True)

                @pl.when(j0 < n_super - 1)
                def _prefetch():
                    fire_idx(j0 + 1, b)

            return carry

        lax.fori_loop(0, n_super, body, 0)
        for b in range(NB):
            wait_scat(b)
        plsc.subcore_barrier()
        pltpu.sync_copy(
            acc.at[pl.ds(sid * out_rows, out_rows)],
            out_hbm.at[cid, pl.ds(sid * out_rows, out_rows)],
        )

    return k(h, src_pad, dst_pad, zeros_blk)


def _vspec():
    return pl.BlockSpec(memory_space=pltpu.VMEM)


def _layer0_body(x_ref, w1_ref, a1_ref, c1_ref, w2_ref, a2_ref, c2_ref,
                 wl_ref, bl_ref, batch_ref, h_ref, out_ref):
    x = x_ref[...]
    h = jnp.maximum(
        jnp.dot(x, w1_ref[...], precision="highest") * a1_ref[...] + c1_ref[...], 0.0)
    h = jnp.maximum(
        jnp.dot(h, w2_ref[...], precision="highest") * a2_ref[...] + c2_ref[...], 0.0)
    h_ref[...] = h
    iota = lax.broadcasted_iota(jnp.int32, (G, N), 0)
    onehot = (batch_ref[...] == iota).astype(jnp.float32)
    pooled = jnp.dot(onehot, h, precision="highest")
    counts = jnp.sum(onehot, axis=1, keepdims=True)
    out_ref[...] = (jnp.dot(pooled, wl_ref[...], precision="highest")
                    + counts * bl_ref[...])


def _layerl_body(h_in_ref, agg_ref, epsp_ref, w1_ref, a1_ref, c1_ref,
                 w2_ref, a2_ref, c2_ref, wl_ref, bl_ref, batch_ref, acc_ref,
                 h_ref, out_ref):
    hin = h_in_ref[...] * epsp_ref[0, 0] + agg_ref[0, :N, :]
    if agg_ref.shape[0] > 1:
        hin = hin + agg_ref[1, :N, :]
    h = jnp.maximum(
        jnp.dot(hin, w1_ref[...], precision="highest") * a1_ref[...] + c1_ref[...], 0.0)
    h = jnp.maximum(
        jnp.dot(h, w2_ref[...], precision="highest") * a2_ref[...] + c2_ref[...], 0.0)
    h_ref[...] = h
    iota = lax.broadcasted_iota(jnp.int32, (G, N), 0)
    onehot = (batch_ref[...] == iota).astype(jnp.float32)
    pooled = jnp.dot(onehot, h, precision="highest")
    out_ref[...] = (acc_ref[...]
                    + jnp.dot(pooled, wl_ref[...], precision="highest")
                    + bl_ref[...])


def _fold_bn(p):
    inv = 1.0 / jnp.sqrt(1.0 + BN_EPS)
    a1 = (p["g1"] * inv)[None, :]
    c1 = (p["b1"] * p["g1"] * inv + p["be1"])[None, :]
    a2 = (p["g2"] * inv)[None, :]
    c2 = (p["b2"] * p["g2"] * inv + p["be2"])[None, :]
    return p["W1"], a1, c1, p["W2"], a2, c2


def _layer0(x, params, batch2):
    w1, a1, c1, w2, a2, c2 = _fold_bn(params["first_h"])
    lin = params["linears"][0]
    return pl.pallas_call(
        _layer0_body,
        out_shape=[
            jax.ShapeDtypeStruct((N, H), jnp.float32),
            jax.ShapeDtypeStruct((G, T), jnp.float32),
        ],
        in_specs=[_vspec()] * 10,
        out_specs=[_vspec(), _vspec()],
    )(x, w1, a1, c1, w2, a2, c2, lin["W"], lin["b"][None, :], batch2)


def _layerl(h, agg, out_acc, conv, lin, batch2):
    w1, a1, c1, w2, a2, c2 = _fold_bn(conv["nn"])
    epsp = (1.0 + conv["eps"]).reshape(1, 1).astype(jnp.float32)
    return pl.pallas_call(
        _layerl_body,
        out_shape=[
            jax.ShapeDtypeStruct((N, H), jnp.float32),
            jax.ShapeDtypeStruct((G, T), jnp.float32),
        ],
        in_specs=([_vspec(), _vspec(), pl.BlockSpec(memory_space=pltpu.SMEM)]
                  + [_vspec()] * 10),
        out_specs=[_vspec(), _vspec()],
    )(h, agg, epsp, w1, a1, c1, w2, a2, c2, lin["W"], lin["b"][None, :],
      batch2, out_acc)


def kernel(x, edge_index, batch, params):
    e = edge_index.shape[1]
    e_pad = NS * EDGE_CHUNK * NB * (S0 + S1)
    assert e_pad >= e
    pad = e_pad - e
    src_pad = jnp.concatenate(
        [edge_index[0], jnp.zeros((pad,), jnp.int32)])
    dst_pad = jnp.concatenate(
        [edge_index[1], jnp.full((pad,), N_ACC - 1, jnp.int32)])
    zeros_blk = jnp.zeros((ZROWS, H), jnp.float32)
    batch2 = batch[None, :]

    h, out = _layer0(x, params, batch2)
    for l in range(L):
        agg = _edge_agg(h, src_pad, dst_pad, zeros_blk)
        h, out = _layerl(h, agg, out, params["convs"][l],
                         params["linears"][l + 1], batch2)
    return jax.nn.sigmoid(out)
